# Initial kernel scaffold; baseline (speedup 1.0000x reference)
import jax
import jax.numpy as jnp
from jax.experimental import pallas as pl


def kernel(x, edge_index, edge_attr, u, batch, W, b):
    raise NotImplementedError("write your pallas kernel here")


# speedup vs baseline: 28.7550x; 28.7550x over previous
import functools

import jax
import jax.numpy as jnp
from jax import lax
from jax.experimental import pallas as pl
from jax.experimental.pallas import tpu as pltpu

N_NODES = 100000
N_EDGES = 3200000
D_FEAT = 128
D_EDGE = 16
N_GRAPHS = 64
D_U = 64
OUT_DIM = 128

BN = 2000
NB = N_NODES // BN
BE = 16000
EB = N_EDGES // BE


def _body(batch_ref, x_ref, eidx_ref, eattr_ref, u_ref, W_ref, b_ref,
          out_ref, acc_node, acc_edge, starts_ref, ends_ref):
    t = pl.program_id(0)

    @pl.when(t == 0)
    def _init():
        acc_node[...] = jnp.zeros_like(acc_node)
        acc_edge[...] = jnp.zeros_like(acc_edge)
        starts_ref[...] = jnp.zeros_like(starts_ref)
        ends_ref[...] = jnp.zeros_like(ends_ref)

    @pl.when(t < NB)
    def _node():
        b_blk = batch_ref[0, 0, :].astype(jnp.int32)[None, :]
        g = lax.broadcasted_iota(jnp.int32, (N_GRAPHS, 1), 0)
        lt = (b_blk < g).astype(jnp.int32)
        le = (b_blk <= g).astype(jnp.int32)
        onehot_t = (le - lt).astype(jnp.float32)
        starts_ref[...] += jnp.sum(lt, axis=1, keepdims=True)
        ends_ref[...] += jnp.sum(le, axis=1, keepdims=True)
        acc_node[...] += jnp.dot(onehot_t, x_ref[...],
                                 preferred_element_type=jnp.float32)

    @pl.when((t >= NB) & (t < NB + EB))
    def _edge():
        idx = eidx_ref[0, 0, 0, :].astype(jnp.int32)[None, :]
        s = starts_ref[...]
        e = ends_ref[...]
        onehot_t = ((idx >= s) & (idx < e)).astype(jnp.float32)
        acc_edge[...] += jnp.dot(onehot_t, eattr_ref[...],
                                 preferred_element_type=jnp.float32)

    @pl.when(t == NB + EB)
    def _final():
        inp = jnp.concatenate(
            [acc_node[...], acc_edge[...], u_ref[...]], axis=-1)
        out = jnp.dot(inp, W_ref[...], preferred_element_type=jnp.float32)
        out_ref[...] = jnp.maximum(out + b_ref[...], 0.0)


@jax.jit
def kernel(x, edge_index, edge_attr, u, batch, W, b):
    batch3 = batch.astype(jnp.int32).reshape(NB, 1, BN)
    eidx4 = edge_index.astype(jnp.int32).reshape(2, EB, 1, BE)
    b2 = b.reshape(1, OUT_DIM)

    grid = (NB + EB + 1,)
    out = pl.pallas_call(
        _body,
        grid=grid,
        in_specs=[
            pl.BlockSpec((1, 1, BN), lambda t: (jnp.minimum(t, NB - 1), 0, 0)),
            pl.BlockSpec((BN, D_FEAT), lambda t: (jnp.minimum(t, NB - 1), 0)),
            pl.BlockSpec((1, 1, 1, BE),
                         lambda t: (0, jnp.clip(t - NB, 0, EB - 1), 0, 0)),
            pl.BlockSpec((BE, D_EDGE),
                         lambda t: (jnp.clip(t - NB, 0, EB - 1), 0)),
            pl.BlockSpec((N_GRAPHS, D_U), lambda t: (0, 0)),
            pl.BlockSpec((D_FEAT + D_EDGE + D_U, OUT_DIM), lambda t: (0, 0)),
            pl.BlockSpec((1, OUT_DIM), lambda t: (0, 0)),
        ],
        out_specs=pl.BlockSpec((N_GRAPHS, OUT_DIM), lambda t: (0, 0)),
        out_shape=jax.ShapeDtypeStruct((N_GRAPHS, OUT_DIM), jnp.float32),
        scratch_shapes=[
            pltpu.VMEM((N_GRAPHS, D_FEAT), jnp.float32),
            pltpu.VMEM((N_GRAPHS, D_EDGE), jnp.float32),
            pltpu.VMEM((N_GRAPHS, 1), jnp.int32),
            pltpu.VMEM((N_GRAPHS, 1), jnp.int32),
        ],
    )(batch3, x, eidx4, edge_attr, u, W, b2)
    return out
